# baseline (speedup 1.0000x reference)
import functools
import jax
import jax.numpy as jnp
from jax import lax
from jax.experimental import pallas as pl
from jax.experimental.pallas import tpu as pltpu
from jax.experimental.pallas import tpu_sc as plsc

_N_SRC = 64
_N_TGT = 128
_E = _N_SRC * _N_SRC
_BN_EPS = 1e-3

_NC = 2
_NS = 16
_NW = _NC * _NS
_EPW = _E // _NW
_ZPW = _E // _NS


def _sc_scatter(src_hbm, dst_hbm, attr_hbm, s_out, cnt_out,
                src_v, dst_v, attr_v, idx_v, ones_v, zer_v, s_sh, cnt_sh):
    c = lax.axis_index("c")
    s = lax.axis_index("s")
    base = (c * _NS + s) * _EPW
    pltpu.sync_copy(src_hbm.at[pl.ds(base, _EPW)], src_v)
    pltpu.sync_copy(dst_hbm.at[pl.ds(base, _EPW)], dst_v)
    pltpu.sync_copy(attr_hbm.at[pl.ds(base, _EPW)], attr_v)

    zeros16 = jnp.zeros((16,), jnp.float32)
    for j in range(_ZPW // 16):
        zer_v[pl.ds(j * 16, 16)] = zeros16
    for j in range(_EPW // 16):
        sl = pl.ds(j * 16, 16)
        ones_v[sl] = zeros16 + 1.0
        idx_v[sl] = dst_v[sl] * _N_SRC + src_v[sl]

    pltpu.sync_copy(zer_v, s_sh.at[pl.ds(s * _ZPW, _ZPW)])

    @pl.when(s == 0)
    def _():
        pltpu.sync_copy(zer_v.at[pl.ds(0, _N_SRC)], cnt_sh)

    plsc.subcore_barrier()
    pltpu.sync_copy(attr_v, s_sh.at[idx_v], add=True)
    pltpu.sync_copy(ones_v, cnt_sh.at[dst_v], add=True)
    plsc.subcore_barrier()

    @pl.when(s == 0)
    def _():
        pltpu.sync_copy(s_sh, s_out.at[c])
        pltpu.sync_copy(cnt_sh, cnt_out.at[c])


def _dense_kernel(s2_ref, cnt2_ref, w1_ref, root1_ref, bias1_ref,
                  g1_ref, be1_ref, rm1_ref, rv1_ref,
                  w3_ref, root3_ref, bias3_ref,
                  g3_ref, be3_ref, rm3_ref, rv3_ref,
                  x_ref, out_ref):
    f32 = jnp.float32
    S = s2_ref[0] + s2_ref[1]
    cnt = cnt2_ref[0] + cnt2_ref[1]
    inv_cnt = 1.0 / jnp.maximum(cnt, 1.0)

    x = x_ref[...]
    wr1 = jax.nn.relu(w1_ref[...])
    g1 = jnp.dot(S, x, preferred_element_type=f32)
    h1 = jnp.dot(g1, wr1, preferred_element_type=f32) * inv_cnt
    h1 = h1 + jnp.dot(x, root1_ref[...], preferred_element_type=f32) + bias1_ref[...]
    h1 = g1_ref[...] * (h1 - rm1_ref[...]) * lax.rsqrt(rv1_ref[...] + _BN_EPS) + be1_ref[...]
    x1 = jax.nn.sigmoid(h1)

    wr3 = jax.nn.relu(w3_ref[...])
    g3 = jnp.dot(S, x1, preferred_element_type=f32)
    h3 = jnp.dot(g3, wr3, preferred_element_type=f32) * inv_cnt
    h3 = h3 + jnp.dot(x1, root3_ref[...], preferred_element_type=f32) + bias3_ref[...]
    h3 = g3_ref[...] * (h3 - rm3_ref[...]) * lax.rsqrt(rv3_ref[...] + _BN_EPS) + be3_ref[...]
    x3 = jax.nn.sigmoid(h3)

    x4 = lax.dot_general(x3, x3, (((0,), (0,)), ((), ())),
                         preferred_element_type=f32)
    x4 = x4 / jnp.max(x4)
    r = lax.broadcasted_iota(jnp.int32, (_N_TGT, _N_TGT), 0)
    cc = lax.broadcasted_iota(jnp.int32, (_N_TGT, _N_TGT), 1)
    out_ref[...] = jnp.where(r == cc, 1.0, x4)


@functools.partial(
    pl.kernel,
    out_type=(jax.ShapeDtypeStruct((_NC, _E), jnp.float32),
              jax.ShapeDtypeStruct((_NC, _N_SRC), jnp.float32)),
    mesh=plsc.VectorSubcoreMesh(core_axis_name="c", subcore_axis_name="s"),
    scratch_types=[
        pltpu.VMEM((_EPW,), jnp.int32),
        pltpu.VMEM((_EPW,), jnp.int32),
        pltpu.VMEM((_EPW,), jnp.float32),
        pltpu.VMEM((_EPW,), jnp.int32),
        pltpu.VMEM((_EPW,), jnp.float32),
        pltpu.VMEM((_ZPW,), jnp.float32),
        pltpu.VMEM_SHARED((_E,), jnp.float32),
        pltpu.VMEM_SHARED((_N_SRC,), jnp.float32),
    ],
)
def _sc_build_adjacency(src_hbm, dst_hbm, attr_hbm, s_out, cnt_out, *scratch):
    _sc_scatter(src_hbm, dst_hbm, attr_hbm, s_out, cnt_out, *scratch)


def kernel(x, edge_index, edge_attr, W_nn1, b_nn1, root1, bias1, gamma1, beta1,
           rm1, rv1, W_nn3, b_nn3, root3, bias3, gamma3, beta3, rm3, rv3):
    ei = edge_index.astype(jnp.int32)
    s_parts, cnt_parts = _sc_build_adjacency(ei[0], ei[1], edge_attr.reshape(_E))

    w1 = W_nn1.reshape(_N_SRC, _N_SRC)
    w3 = W_nn3.reshape(_N_SRC, _N_TGT)
    row = lambda v: v.reshape(1, -1)
    return pl.pallas_call(
        _dense_kernel,
        out_shape=jax.ShapeDtypeStruct((_N_TGT, _N_TGT), jnp.float32),
    )(s_parts.reshape(_NC, _N_SRC, _N_SRC), cnt_parts.reshape(_NC, _N_SRC, 1),
      w1, root1, row(bias1),
      row(gamma1), row(beta1), row(rm1), row(rv1),
      w3, root3, row(bias3),
      row(gamma3), row(beta3), row(rm3), row(rv3),
      x)


# speedup vs baseline: 1.0395x; 1.0395x over previous
import functools
import jax
import jax.numpy as jnp
from jax import lax
from jax.experimental import pallas as pl
from jax.experimental.pallas import tpu as pltpu
from jax.experimental.pallas import tpu_sc as plsc

_N_SRC = 64
_N_TGT = 128
_E = _N_SRC * _N_SRC
_BN_EPS = 1e-3

_NC = 2
_NS = 16
_NW = _NC * _NS
_EPW = _E // _NW
_ZPW = _E // _NS


def _sc_scatter(ei_hbm, attr_hbm, s_out, cnt_out,
                src_v, dst_v, attr_v, idx_v, ones_v, zer_v, s_sh, cnt_sh,
                sem_in, sem_z, sem_s1, sem_s2):
    c = lax.axis_index("c")
    s = lax.axis_index("s")
    base = (c * _NS + s) * _EPW
    cp_src = pltpu.async_copy(ei_hbm.at[0, pl.ds(base, _EPW)], src_v, sem_in)
    cp_dst = pltpu.async_copy(ei_hbm.at[1, pl.ds(base, _EPW)], dst_v, sem_in)
    cp_att = pltpu.async_copy(attr_hbm.at[pl.ds(base, _EPW)], attr_v, sem_in)

    zeros16 = jnp.zeros((16,), jnp.float32)
    for j in range(_ZPW // 16):
        zer_v[pl.ds(j * 16, 16)] = zeros16
    cp_z = pltpu.async_copy(zer_v, s_sh.at[pl.ds(s * _ZPW, _ZPW)], sem_z)
    for j in range(_EPW // 16):
        ones_v[pl.ds(j * 16, 16)] = zeros16 + 1.0

    cp_src.wait()
    cp_dst.wait()
    for j in range(_EPW // 16):
        sl = pl.ds(j * 16, 16)
        idx_v[sl] = dst_v[sl] * _N_SRC + src_v[sl]
    cp_att.wait()
    cp_z.wait()

    @pl.when(s == 0)
    def _():
        pltpu.sync_copy(zer_v.at[pl.ds(0, _N_SRC)], cnt_sh)

    plsc.subcore_barrier()
    sc1 = pltpu.async_copy(attr_v, s_sh.at[idx_v], sem_s1, add=True)
    sc2 = pltpu.async_copy(ones_v, cnt_sh.at[dst_v], sem_s2, add=True)
    sc1.wait()
    sc2.wait()
    plsc.subcore_barrier()

    pltpu.sync_copy(s_sh.at[pl.ds(s * _ZPW, _ZPW)], s_out.at[c, pl.ds(s * _ZPW, _ZPW)])

    @pl.when(s == 0)
    def _():
        pltpu.sync_copy(cnt_sh, cnt_out.at[c])


def _dense_kernel(s2_ref, cnt2_ref, w1_ref, root1_ref, bias1_ref,
                  g1_ref, be1_ref, rm1_ref, rv1_ref,
                  w3_ref, root3_ref, bias3_ref,
                  g3_ref, be3_ref, rm3_ref, rv3_ref,
                  x_ref, out_ref):
    f32 = jnp.float32
    S = s2_ref[0] + s2_ref[1]
    cnt = cnt2_ref[0] + cnt2_ref[1]
    inv_cnt = 1.0 / jnp.maximum(cnt, 1.0)

    x = x_ref[...]
    wr1 = jax.nn.relu(w1_ref[...])
    g1 = jnp.dot(S, x, preferred_element_type=f32)
    h1 = jnp.dot(g1, wr1, preferred_element_type=f32) * inv_cnt
    h1 = h1 + jnp.dot(x, root1_ref[...], preferred_element_type=f32) + bias1_ref[...]
    h1 = g1_ref[...] * (h1 - rm1_ref[...]) * lax.rsqrt(rv1_ref[...] + _BN_EPS) + be1_ref[...]
    x1 = jax.nn.sigmoid(h1)

    wr3 = jax.nn.relu(w3_ref[...])
    g3 = jnp.dot(S, x1, preferred_element_type=f32)
    h3 = jnp.dot(g3, wr3, preferred_element_type=f32) * inv_cnt
    h3 = h3 + jnp.dot(x1, root3_ref[...], preferred_element_type=f32) + bias3_ref[...]
    h3 = g3_ref[...] * (h3 - rm3_ref[...]) * lax.rsqrt(rv3_ref[...] + _BN_EPS) + be3_ref[...]
    x3 = jax.nn.sigmoid(h3)

    x4 = lax.dot_general(x3, x3, (((0,), (0,)), ((), ())),
                         preferred_element_type=f32)
    x4 = x4 / jnp.max(x4)
    r = lax.broadcasted_iota(jnp.int32, (_N_TGT, _N_TGT), 0)
    cc = lax.broadcasted_iota(jnp.int32, (_N_TGT, _N_TGT), 1)
    out_ref[...] = jnp.where(r == cc, 1.0, x4)


@functools.partial(
    pl.kernel,
    out_type=(jax.ShapeDtypeStruct((_NC, _E), jnp.float32),
              jax.ShapeDtypeStruct((_NC, _N_SRC), jnp.float32)),
    mesh=plsc.VectorSubcoreMesh(core_axis_name="c", subcore_axis_name="s"),
    scratch_types=[
        pltpu.VMEM((_EPW,), jnp.int32),
        pltpu.VMEM((_EPW,), jnp.int32),
        pltpu.VMEM((_EPW,), jnp.float32),
        pltpu.VMEM((_EPW,), jnp.int32),
        pltpu.VMEM((_EPW,), jnp.float32),
        pltpu.VMEM((_ZPW,), jnp.float32),
        pltpu.VMEM_SHARED((_E,), jnp.float32),
        pltpu.VMEM_SHARED((_N_SRC,), jnp.float32),
        pltpu.SemaphoreType.DMA,
        pltpu.SemaphoreType.DMA,
        pltpu.SemaphoreType.DMA,
        pltpu.SemaphoreType.DMA,
    ],
)
def _sc_build_adjacency(ei_hbm, attr_hbm, s_out, cnt_out, *scratch):
    _sc_scatter(ei_hbm, attr_hbm, s_out, cnt_out, *scratch)


def kernel(x, edge_index, edge_attr, W_nn1, b_nn1, root1, bias1, gamma1, beta1,
           rm1, rv1, W_nn3, b_nn3, root3, bias3, gamma3, beta3, rm3, rv3):
    ei = edge_index.astype(jnp.int32)
    s_parts, cnt_parts = _sc_build_adjacency(ei, edge_attr.reshape(_E))

    w1 = W_nn1.reshape(_N_SRC, _N_SRC)
    w3 = W_nn3.reshape(_N_SRC, _N_TGT)
    row = lambda v: v.reshape(1, -1)
    return pl.pallas_call(
        _dense_kernel,
        out_shape=jax.ShapeDtypeStruct((_N_TGT, _N_TGT), jnp.float32),
    )(s_parts.reshape(_NC, _N_SRC, _N_SRC), cnt_parts.reshape(_NC, _N_SRC, 1),
      w1, root1, row(bias1),
      row(gamma1), row(beta1), row(rm1), row(rv1),
      w3, root3, row(bias3),
      row(gamma3), row(beta3), row(rm3), row(rv3),
      x)


# speedup vs baseline: 1.0632x; 1.0227x over previous
import functools
import jax
import jax.numpy as jnp
from jax import lax
from jax.experimental import pallas as pl
from jax.experimental.pallas import tpu as pltpu
from jax.experimental.pallas import tpu_sc as plsc

_N_SRC = 64
_N_TGT = 128
_E = _N_SRC * _N_SRC
_BN_EPS = 1e-3

_NC = 2
_NS = 16
_NW = _NC * _NS
_EPW = _E // _NW
_ZPW = _E // _NS


def _sc_scatter(ei_hbm, attr_hbm, s_out, cnt_out,
                src_v, dst_v, attr_v, idx_v, ones_v, zer_v, cnt_v,
                s_sh, cnt_sh,
                sem_in, sem_z, sem_s1, sem_s2):
    c = lax.axis_index("c")
    s = lax.axis_index("s")
    base = (c * _NS + s) * _EPW
    cp_src = pltpu.async_copy(ei_hbm.at[0, pl.ds(base, _EPW)], src_v, sem_in)
    cp_dst = pltpu.async_copy(ei_hbm.at[1, pl.ds(base, _EPW)], dst_v, sem_in)
    cp_att = pltpu.async_copy(attr_hbm.at[pl.ds(base, _EPW)], attr_v, sem_in)

    zeros16 = jnp.zeros((16,), jnp.float32)
    for j in range(_ZPW // 16):
        zer_v[pl.ds(j * 16, 16)] = zeros16
    cp_z = pltpu.async_copy(zer_v, s_sh.at[pl.ds(s * _ZPW, _ZPW)], sem_z)
    for j in range(_EPW // 16):
        ones_v[pl.ds(j * 16, 16)] = zeros16 + 1.0

    cp_src.wait()
    cp_dst.wait()
    for j in range(_EPW // 16):
        sl = pl.ds(j * 16, 16)
        idx_v[sl] = dst_v[sl] * _N_SRC + src_v[sl]
    cp_att.wait()
    cp_z.wait()

    @pl.when(s == 0)
    def _():
        pltpu.sync_copy(zer_v.at[pl.ds(0, _N_SRC)], cnt_sh)

    plsc.subcore_barrier()
    sc1 = pltpu.async_copy(attr_v, s_sh.at[idx_v], sem_s1, add=True)
    sc2 = pltpu.async_copy(ones_v, cnt_sh.at[dst_v], sem_s2, add=True)
    sc1.wait()
    sc2.wait()
    plsc.subcore_barrier()

    pltpu.sync_copy(s_sh.at[pl.ds(s * _ZPW, _ZPW)], zer_v)
    pltpu.sync_copy(zer_v, s_out.at[c, pl.ds(s * _ZPW, _ZPW)])

    @pl.when(s == 0)
    def _():
        pltpu.sync_copy(cnt_sh, cnt_v)
        pltpu.sync_copy(cnt_v, cnt_out.at[c])


def _dense_kernel(s2_ref, cnt2_ref, w1_ref, root1_ref, bias1_ref,
                  g1_ref, be1_ref, rm1_ref, rv1_ref,
                  w3_ref, root3_ref, bias3_ref,
                  g3_ref, be3_ref, rm3_ref, rv3_ref,
                  x_ref, out_ref):
    f32 = jnp.float32
    S = s2_ref[0] + s2_ref[1]
    cnt = cnt2_ref[0] + cnt2_ref[1]
    inv_cnt = 1.0 / jnp.maximum(cnt, 1.0)

    x = x_ref[...]
    wr1 = jax.nn.relu(w1_ref[...])
    g1 = jnp.dot(S, x, preferred_element_type=f32)
    h1 = jnp.dot(g1, wr1, preferred_element_type=f32) * inv_cnt
    h1 = h1 + jnp.dot(x, root1_ref[...], preferred_element_type=f32) + bias1_ref[...]
    h1 = g1_ref[...] * (h1 - rm1_ref[...]) * lax.rsqrt(rv1_ref[...] + _BN_EPS) + be1_ref[...]
    x1 = jax.nn.sigmoid(h1)

    wr3 = jax.nn.relu(w3_ref[...])
    g3 = jnp.dot(S, x1, preferred_element_type=f32)
    h3 = jnp.dot(g3, wr3, preferred_element_type=f32) * inv_cnt
    h3 = h3 + jnp.dot(x1, root3_ref[...], preferred_element_type=f32) + bias3_ref[...]
    h3 = g3_ref[...] * (h3 - rm3_ref[...]) * lax.rsqrt(rv3_ref[...] + _BN_EPS) + be3_ref[...]
    x3 = jax.nn.sigmoid(h3)

    x4 = lax.dot_general(x3, x3, (((0,), (0,)), ((), ())),
                         preferred_element_type=f32)
    x4 = x4 / jnp.max(x4)
    r = lax.broadcasted_iota(jnp.int32, (_N_TGT, _N_TGT), 0)
    cc = lax.broadcasted_iota(jnp.int32, (_N_TGT, _N_TGT), 1)
    out_ref[...] = jnp.where(r == cc, 1.0, x4)


@functools.partial(
    pl.kernel,
    out_type=[
        jax.ShapeDtypeStruct((_NC, _E), jnp.float32),
        jax.ShapeDtypeStruct((_NC, _N_SRC), jnp.float32),
    ],
    mesh=plsc.VectorSubcoreMesh(core_axis_name="c", subcore_axis_name="s"),
    scratch_types=[
        pltpu.VMEM((_EPW,), jnp.int32),
        pltpu.VMEM((_EPW,), jnp.int32),
        pltpu.VMEM((_EPW,), jnp.float32),
        pltpu.VMEM((_EPW,), jnp.int32),
        pltpu.VMEM((_EPW,), jnp.float32),
        pltpu.VMEM((_ZPW,), jnp.float32),
        pltpu.VMEM((_N_SRC,), jnp.float32),
        pltpu.VMEM_SHARED((_E,), jnp.float32),
        pltpu.VMEM_SHARED((_N_SRC,), jnp.float32),
        pltpu.SemaphoreType.DMA,
        pltpu.SemaphoreType.DMA,
        pltpu.SemaphoreType.DMA,
        pltpu.SemaphoreType.DMA,
    ],
)
def _sc_build_adjacency(ei_hbm, attr_hbm, s_out, cnt_out, *scratch):
    _sc_scatter(ei_hbm, attr_hbm, s_out, cnt_out, *scratch)


def kernel(x, edge_index, edge_attr, W_nn1, b_nn1, root1, bias1, gamma1, beta1,
           rm1, rv1, W_nn3, b_nn3, root3, bias3, gamma3, beta3, rm3, rv3):
    ei = edge_index.astype(jnp.int32)
    s_parts, cnt_parts = _sc_build_adjacency(ei, edge_attr.reshape(_E))

    w1 = W_nn1.reshape(_N_SRC, _N_SRC)
    w3 = W_nn3.reshape(_N_SRC, _N_TGT)
    row = lambda v: v.reshape(1, -1)
    return pl.pallas_call(
        _dense_kernel,
        out_shape=jax.ShapeDtypeStruct((_N_TGT, _N_TGT), jnp.float32),
    )(s_parts.reshape(_NC, _N_SRC, _N_SRC), cnt_parts.reshape(_NC, _N_SRC, 1),
      w1, root1, row(bias1),
      row(gamma1), row(beta1), row(rm1), row(rv1),
      w3, root3, row(bias3),
      row(gamma3), row(beta3), row(rm3), row(rv3),
      x)


# speedup vs baseline: 1.1951x; 1.1241x over previous
import functools
import jax
import jax.numpy as jnp
from jax import lax
from jax.experimental import pallas as pl
from jax.experimental.pallas import tpu as pltpu
from jax.experimental.pallas import tpu_sc as plsc

_N_SRC = 64
_N_TGT = 128
_E = _N_SRC * _N_SRC
_BN_EPS = 1e-3

_NC = 2
_NS = 16
_NW = _NC * _NS
_EPW = _E // _NW
_LP = 128
_SACC = _N_SRC * _LP
_ZPW = _SACC // _NS


def _sc_scatter(ei_hbm, attr_hbm, s_out, cnt_out,
                src_v, dst_v, attr_v, idx_v, ones_v, zer_v, cnt_v,
                s_sh, cnt_sh,
                sem_in, sem_z, sem_s1, sem_s2):
    c = lax.axis_index("c")
    s = lax.axis_index("s")
    base = (c * _NS + s) * _EPW
    cp_src = pltpu.async_copy(ei_hbm.at[0, pl.ds(base, _EPW)], src_v, sem_in)
    cp_dst = pltpu.async_copy(ei_hbm.at[1, pl.ds(base, _EPW)], dst_v, sem_in)
    cp_att = pltpu.async_copy(attr_hbm.at[pl.ds(base, _EPW)], attr_v, sem_in)

    zeros16 = jnp.zeros((16,), jnp.float32)
    for j in range(_ZPW // 16):
        zer_v[pl.ds(j * 16, 16)] = zeros16
    cp_z = pltpu.async_copy(zer_v, s_sh.at[pl.ds(s * _ZPW, _ZPW)], sem_z)
    for j in range(_EPW // 16):
        ones_v[pl.ds(j * 16, 16)] = zeros16 + 1.0

    cp_src.wait()
    cp_dst.wait()
    for j in range(_EPW // 16):
        sl = pl.ds(j * 16, 16)
        idx_v[sl] = dst_v[sl] * _LP + src_v[sl]
    cp_att.wait()
    cp_z.wait()

    @pl.when(s == 0)
    def _():
        pltpu.sync_copy(zer_v.at[pl.ds(0, _N_SRC)], cnt_sh)

    plsc.subcore_barrier()
    sc1 = pltpu.async_copy(attr_v, s_sh.at[idx_v], sem_s1, add=True)
    sc2 = pltpu.async_copy(ones_v, cnt_sh.at[dst_v], sem_s2, add=True)
    sc1.wait()
    sc2.wait()
    plsc.subcore_barrier()

    pltpu.sync_copy(s_sh.at[pl.ds(s * _ZPW, _ZPW)], zer_v)
    rows_per_sub = _ZPW // _LP
    for r in range(rows_per_sub):
        pltpu.sync_copy(zer_v.at[pl.ds(r * _LP, _LP)],
                        s_out.at[c, s * rows_per_sub + r])

    @pl.when(s == 0)
    def _():
        pltpu.sync_copy(cnt_sh, cnt_v)
        pltpu.sync_copy(cnt_v, cnt_out.at[c])


def _dense_kernel(s2_ref, cnt2_ref, w1_ref, root1_ref, bias1_ref,
                  g1_ref, be1_ref, rm1_ref, rv1_ref,
                  w3_ref, root3_ref, bias3_ref,
                  g3_ref, be3_ref, rm3_ref, rv3_ref,
                  x_ref, out_ref):
    f32 = jnp.float32
    S = (s2_ref[0] + s2_ref[1])[:, :_N_SRC]
    cnt = cnt2_ref[0:1] + cnt2_ref[1:2]
    inv_cnt = 1.0 / jnp.maximum(cnt, 1.0)
    r64 = lax.broadcasted_iota(jnp.int32, (_N_SRC, _N_SRC), 0)
    c64 = lax.broadcasted_iota(jnp.int32, (_N_SRC, _N_SRC), 1)
    dmat = jnp.where(r64 == c64, jnp.broadcast_to(inv_cnt, (_N_SRC, _N_SRC)), 0.0)
    Sn = jnp.dot(dmat, S, preferred_element_type=f32)

    x = x_ref[...]
    wr1 = jax.nn.relu(w1_ref[...])
    g1 = jnp.dot(Sn, x, preferred_element_type=f32)
    h1 = jnp.dot(g1, wr1, preferred_element_type=f32)
    h1 = h1 + jnp.dot(x, root1_ref[...], preferred_element_type=f32) + bias1_ref[...]
    h1 = g1_ref[...] * (h1 - rm1_ref[...]) * lax.rsqrt(rv1_ref[...] + _BN_EPS) + be1_ref[...]
    x1 = jax.nn.sigmoid(h1)

    wr3 = jax.nn.relu(w3_ref[...])
    g3 = jnp.dot(Sn, x1, preferred_element_type=f32)
    h3 = jnp.dot(g3, wr3, preferred_element_type=f32)
    h3 = h3 + jnp.dot(x1, root3_ref[...], preferred_element_type=f32) + bias3_ref[...]
    h3 = g3_ref[...] * (h3 - rm3_ref[...]) * lax.rsqrt(rv3_ref[...] + _BN_EPS) + be3_ref[...]
    x3 = jax.nn.sigmoid(h3)

    x4 = lax.dot_general(x3, x3, (((0,), (0,)), ((), ())),
                         preferred_element_type=f32)
    x4 = x4 / jnp.max(x4)
    r = lax.broadcasted_iota(jnp.int32, (_N_TGT, _N_TGT), 0)
    cc = lax.broadcasted_iota(jnp.int32, (_N_TGT, _N_TGT), 1)
    out_ref[...] = jnp.where(r == cc, 1.0, x4)


@functools.partial(
    pl.kernel,
    out_type=[
        jax.ShapeDtypeStruct((_NC, _N_SRC, _LP), jnp.float32),
        jax.ShapeDtypeStruct((_NC, _N_SRC), jnp.float32),
    ],
    mesh=plsc.VectorSubcoreMesh(core_axis_name="c", subcore_axis_name="s"),
    scratch_types=[
        pltpu.VMEM((_EPW,), jnp.int32),
        pltpu.VMEM((_EPW,), jnp.int32),
        pltpu.VMEM((_EPW,), jnp.float32),
        pltpu.VMEM((_EPW,), jnp.int32),
        pltpu.VMEM((_EPW,), jnp.float32),
        pltpu.VMEM((_ZPW,), jnp.float32),
        pltpu.VMEM((_N_SRC,), jnp.float32),
        pltpu.VMEM_SHARED((_SACC,), jnp.float32),
        pltpu.VMEM_SHARED((_N_SRC,), jnp.float32),
        pltpu.SemaphoreType.DMA,
        pltpu.SemaphoreType.DMA,
        pltpu.SemaphoreType.DMA,
        pltpu.SemaphoreType.DMA,
    ],
)
def _sc_build_adjacency(ei_hbm, attr_hbm, s_out, cnt_out, *scratch):
    _sc_scatter(ei_hbm, attr_hbm, s_out, cnt_out, *scratch)


def kernel(x, edge_index, edge_attr, W_nn1, b_nn1, root1, bias1, gamma1, beta1,
           rm1, rv1, W_nn3, b_nn3, root3, bias3, gamma3, beta3, rm3, rv3):
    ei = edge_index.astype(jnp.int32)
    s_parts, cnt_parts = _sc_build_adjacency(ei, edge_attr.reshape(_E))

    w1 = W_nn1.reshape(_N_SRC, _N_SRC)
    w3 = W_nn3.reshape(_N_SRC, _N_TGT)
    row = lambda v: v.reshape(1, -1)
    return pl.pallas_call(
        _dense_kernel,
        out_shape=jax.ShapeDtypeStruct((_N_TGT, _N_TGT), jnp.float32),
    )(s_parts, cnt_parts,
      w1, root1, row(bias1),
      row(gamma1), row(beta1), row(rm1), row(rv1),
      w3, root3, row(bias3),
      row(gamma3), row(beta3), row(rm3), row(rv3),
      x)
